# baseline (speedup 1.0000x reference)
import functools

import jax
import jax.numpy as jnp
from jax import lax
from jax.experimental import pallas as pl
from jax.experimental.pallas import tpu as pltpu
from jax.experimental.pallas import tpu_sc as plsc

_N, _D = 8192, 768
_PATHS = 8
_NC, _NS, _L = 2, 16, 16
_NW = _NC * _NS
_RPW = _N // _NW
_C = 64
_NCHUNK = _RPW // _C
_ZR = 16
_NPAD = _RPW // _ZR

_mesh = plsc.VectorSubcoreMesh(core_axis_name="c", subcore_axis_name="s")


@functools.partial(
    pl.kernel,
    mesh=_mesh,
    out_type=[jax.ShapeDtypeStruct((_N, _D), jnp.float32)] * 2,
    scratch_types=[
        pltpu.VMEM((_C, _D), jnp.float32),
        pltpu.VMEM((_C, _D), jnp.float32),
        pltpu.VMEM((_ZR, _D), jnp.float32),
        pltpu.SemaphoreType.DMA,
        pltpu.SemaphoreType.DMA,
        pltpu.SemaphoreType.DMA,
    ],
)
def _sc_dispatch(x_hbm, hit_hbm, pad_hbm, buf0, buf1, zbuf, in_sem, out_sem,
                 pad_sem):
    wid = lax.axis_index("s") * _NC + lax.axis_index("c")
    base = wid * _RPW

    zvec = jnp.zeros((_L,), jnp.float32)

    def _zero(k, carry):
        i = k // (_D // _L)
        j = k % (_D // _L)
        zbuf[i, pl.ds(j * _L, _L)] = zvec
        return carry

    lax.fori_loop(0, _ZR * (_D // _L), _zero, 0)

    pads = [
        pltpu.async_copy(zbuf, pad_hbm.at[pl.ds(base + t * _ZR, _ZR), :],
                         pad_sem)
        for t in range(_NPAD)
    ]

    bufs = (buf0, buf1)
    ins = [None] * _NCHUNK
    outs = [None] * _NCHUNK
    ins[0] = pltpu.async_copy(x_hbm.at[pl.ds(base, _C), :], buf0, in_sem)
    for k in range(_NCHUNK):
        if k + 1 < _NCHUNK:
            if k >= 1:
                outs[k - 1].wait()
            ins[k + 1] = pltpu.async_copy(
                x_hbm.at[pl.ds(base + (k + 1) * _C, _C), :],
                bufs[(k + 1) % 2], in_sem)
        ins[k].wait()
        outs[k] = pltpu.async_copy(
            bufs[k % 2], hit_hbm.at[pl.ds(base + k * _C, _C), :], out_sem)
    outs[_NCHUNK - 2].wait()
    outs[_NCHUNK - 1].wait()
    for p in pads:
        p.wait()


def kernel(inputs):
    hit, pad = _sc_dispatch(inputs)
    return (hit,) + (pad,) * (_PATHS - 1)
